# baseline (speedup 1.0000x reference)
import functools

import jax
import jax.numpy as jnp
from jax import lax
from jax.experimental import pallas as pl
from jax.experimental.pallas import tpu as pltpu
from jax.experimental.pallas import tpu_sc as plsc

_WIDTH = 64
_NC = 2
_NS = 16
_NW = _NC * _NS

_IDX_PER_STREAM = 128
_STREAMS_PER_ITER = 8
_CHUNK = _IDX_PER_STREAM * _STREAMS_PER_ITER


def _make_gather(total, width):
    per_w = total // _NW
    n_iter = per_w // _CHUNK
    mesh = plsc.VectorSubcoreMesh(
        core_axis_name="c", subcore_axis_name="s",
        num_cores=_NC, num_subcores=_NS,
    )

    @functools.partial(
        pl.kernel,
        mesh=mesh,
        out_type=jax.ShapeDtypeStruct((total, width), jnp.float32),
        scratch_types=[
            pltpu.VMEM((_STREAMS_PER_ITER, _IDX_PER_STREAM), jnp.int32),
            pltpu.VMEM((_CHUNK, width), jnp.float32),
            pltpu.SemaphoreType.DMA,
        ],
        compiler_params=pltpu.CompilerParams(use_tc_tiling_on_sc=False),
    )
    def gather_kernel(table_hbm, idx_hbm, out_hbm, idx_v, rows_v, sem):
        wid = lax.axis_index("s") * _NC + lax.axis_index("c")
        row_base = wid * per_w

        @pl.loop(0, n_iter)
        def _(g):
            off = row_base + g * _CHUNK
            pltpu.sync_copy(
                idx_hbm.at[pl.ds(pl.multiple_of(off // _IDX_PER_STREAM, 8),
                                 _STREAMS_PER_ITER)],
                idx_v,
            )
            copies = []
            for j in range(_STREAMS_PER_ITER):
                copies.append(pltpu.async_copy(
                    table_hbm.at[idx_v.at[j]],
                    rows_v.at[pl.ds(j * _IDX_PER_STREAM, _IDX_PER_STREAM)],
                    sem,
                ))
            for c in copies:
                c.wait()
            pltpu.sync_copy(rows_v, out_hbm.at[pl.ds(off, _CHUNK)])

    return gather_kernel


@jax.jit
def kernel(tokens, embedding_weight):
    b, s = tokens.shape
    total = b * s
    idx = tokens.reshape(total // _IDX_PER_STREAM, _IDX_PER_STREAM)
    out = _make_gather(total, embedding_weight.shape[1])(
        embedding_weight, idx.astype(jnp.int32))
    return out.reshape(b, s, embedding_weight.shape[1])


# speedup vs baseline: 1.0152x; 1.0152x over previous
import functools

import jax
import jax.numpy as jnp
from jax import lax
from jax.experimental import pallas as pl
from jax.experimental.pallas import tpu as pltpu
from jax.experimental.pallas import tpu_sc as plsc

_NC = 2
_NS = 16
_NW = _NC * _NS

_ROWS_PER_CHUNK = 2
_NBUF = 2
_SPLITS = ((0, 104), (104, 96))


def _make_gather(n_rows, seq, vocab, width):
    rows_per_w = n_rows // _NW
    n_chunks = rows_per_w // _ROWS_PER_CHUNK
    mesh = plsc.VectorSubcoreMesh(
        core_axis_name="c", subcore_axis_name="s",
        num_cores=_NC, num_subcores=_NS,
    )

    @functools.partial(
        pl.kernel,
        mesh=mesh,
        out_type=jax.ShapeDtypeStruct((n_rows, seq, width), jnp.float32),
        scratch_types=[
            pltpu.VMEM((rows_per_w, seq), jnp.int32),
            pltpu.VMEM((_NBUF, _ROWS_PER_CHUNK, seq, width), jnp.float32),
        ] + [pltpu.SemaphoreType.DMA] * _NBUF,
        compiler_params=pltpu.CompilerParams(use_tc_tiling_on_sc=False),
    )
    def gather_kernel(table_hbm, tok_hbm, out_hbm, tok_v, rows_v, *gsems):
        wid = lax.axis_index("s") * _NC + lax.axis_index("c")
        row_base = pl.multiple_of(wid * rows_per_w, 8)

        pltpu.sync_copy(tok_hbm.at[pl.ds(row_base, rows_per_w)], tok_v)

        def chunk_copies(c, b):
            ds = []
            for rr in range(_ROWS_PER_CHUNK):
                row = c * _ROWS_PER_CHUNK + rr
                for (o, n) in _SPLITS:
                    ds.append(pltpu.make_async_copy(
                        table_hbm.at[tok_v.at[row, pl.ds(o, n)]],
                        rows_v.at[b, rr, pl.ds(o, n)],
                        gsems[b],
                    ))
            return ds

        def fire(c, b):
            for d in chunk_copies(c, b):
                d.start()

        def drain(c, b):
            for d in chunk_copies(c, b):
                d.wait()

        def write_out(c, b):
            pltpu.sync_copy(
                rows_v.at[b],
                out_hbm.at[pl.ds(row_base + c * _ROWS_PER_CHUNK,
                                 _ROWS_PER_CHUNK)],
            )

        for b in range(_NBUF):
            fire(b, b)

        @pl.loop(0, n_chunks - _NBUF, step=_NBUF)
        def _(g0):
            for b in range(_NBUF):
                c = g0 + b
                drain(c, b)
                write_out(c, b)
                fire(c + _NBUF, b)

        for b in range(_NBUF):
            c = n_chunks - _NBUF + b
            drain(c, b)
            write_out(c, b)

    return gather_kernel


@jax.jit
def kernel(tokens, embedding_weight):
    n_rows, seq = tokens.shape
    return _make_gather(n_rows, seq, *embedding_weight.shape)(
        embedding_weight, tokens)
